# Initial kernel scaffold; baseline (speedup 1.0000x reference)
import jax
import jax.numpy as jnp
from jax.experimental import pallas as pl


def kernel(x_transaction, x_card, x_device, tx_time_decay, params, edge_tc, edge_td, edge_ct, edge_dt):
    raise NotImplementedError("write your pallas kernel here")


# speedup vs baseline: 8.8173x; 8.8173x over previous
import functools

import numpy as np
import jax
import jax.numpy as jnp
from jax import lax
from jax.experimental import pallas as pl
from jax.experimental.pallas import tpu as pltpu

_TYPES = ["transaction", "card", "device"]
_EDGE_DEFS = {
    "tc": ("transaction", "card"),
    "td": ("transaction", "device"),
    "ct": ("card", "transaction"),
    "dt": ("device", "transaction"),
}
_DST_GROUPS = {"transaction": ["ct", "dt"], "card": ["tc"], "device": ["td"]}
_D_H = 128
_HEADS = 2
_DH = 64
_VW = 144

_INV_SQRT2 = np.float32(1.0 / np.sqrt(2.0))


def _gelu(x):
    return 0.5 * x * (1.0 + lax.erf(x * _INV_SQRT2))


def _mm_body(x_ref, w_ref, b_ref, o_ref):
    o_ref[...] = (
        jnp.dot(x_ref[...], w_ref[...], preferred_element_type=jnp.float32)
        + b_ref[...]
    )


def _mm(x, w, b, bm=2000):
    m, k = x.shape
    n = w.shape[1]
    return pl.pallas_call(
        _mm_body,
        grid=(m // bm,),
        in_specs=[
            pl.BlockSpec((bm, k), lambda i: (i, 0)),
            pl.BlockSpec((k, n), lambda i: (0, 0)),
            pl.BlockSpec((1, n), lambda i: (0, 0)),
        ],
        out_specs=pl.BlockSpec((bm, n), lambda i: (i, 0)),
        out_shape=jax.ShapeDtypeStruct((m, n), jnp.float32),
    )(x, w, b.reshape(1, n))


def _nl_tx_body(x_ref, w_ref, b_ref, d_ref, o_ref):
    h = _gelu(
        jnp.dot(x_ref[...], w_ref[...], preferred_element_type=jnp.float32)
        + b_ref[...]
    )
    o_ref[...] = h * d_ref[...]


def _nl_tx(x, w, b, decay, bm=2000):
    m, k = x.shape
    n = w.shape[1]
    return pl.pallas_call(
        _nl_tx_body,
        grid=(m // bm,),
        in_specs=[
            pl.BlockSpec((bm, k), lambda i: (i, 0)),
            pl.BlockSpec((k, n), lambda i: (0, 0)),
            pl.BlockSpec((1, n), lambda i: (0, 0)),
            pl.BlockSpec((bm, 1), lambda i: (i, 0)),
        ],
        out_specs=pl.BlockSpec((bm, n), lambda i: (i, 0)),
        out_shape=jax.ShapeDtypeStruct((m, n), jnp.float32),
    )(x, w, b.reshape(1, n), decay.reshape(m, 1))


def _nl_scalar_body(x_ref, w_ref, b_ref, o_ref):
    o_ref[...] = _gelu(x_ref[...] * w_ref[...] + b_ref[...])


def _nl_scalar(x, w, b, bm=2000):
    m = x.shape[0]
    n = w.shape[1]
    return pl.pallas_call(
        _nl_scalar_body,
        grid=(m // bm,),
        in_specs=[
            pl.BlockSpec((bm, 1), lambda i: (i, 0)),
            pl.BlockSpec((1, n), lambda i: (0, 0)),
            pl.BlockSpec((1, n), lambda i: (0, 0)),
        ],
        out_specs=pl.BlockSpec((bm, n), lambda i: (i, 0)),
        out_shape=jax.ShapeDtypeStruct((m, n), jnp.float32),
    )(x, w.reshape(1, n), b.reshape(1, n))


def _attn_out_body(agg_ref, h_ref, w_ref, b_ref, beta_ref, o_ref):
    bm = agg_ref.shape[0]
    agg = agg_ref[:, :_D_H]
    s0 = agg_ref[:, _D_H : _D_H + 1]
    s1 = agg_ref[:, _D_H + 1 : _D_H + 2]
    div = (
        jnp.concatenate(
            [jnp.broadcast_to(s0, (bm, _DH)), jnp.broadcast_to(s1, (bm, _DH))],
            axis=1,
        )
        + 1e-16
    )
    g = _gelu(agg / div)
    o = (
        jnp.dot(g, w_ref[...], preferred_element_type=jnp.float32)
        + b_ref[...]
    )
    beta = beta_ref[0, 0]
    out = beta * o + (1.0 - beta) * h_ref[...]
    o_ref[...] = _gelu(out) + h_ref[...]


def _attn_out(agg_raw, h, w, b, beta, bm=2000):
    m = h.shape[0]
    n = w.shape[1]
    return pl.pallas_call(
        _attn_out_body,
        grid=(m // bm,),
        in_specs=[
            pl.BlockSpec((bm, _VW), lambda i: (i, 0)),
            pl.BlockSpec((bm, n), lambda i: (i, 0)),
            pl.BlockSpec((n, n), lambda i: (0, 0)),
            pl.BlockSpec((1, n), lambda i: (0, 0)),
            pl.BlockSpec((1, 1), lambda i: (0, 0)),
        ],
        out_specs=pl.BlockSpec((bm, n), lambda i: (i, 0)),
        out_shape=jax.ShapeDtypeStruct((m, n), jnp.float32),
    )(agg_raw, h, w, b.reshape(1, n), beta.reshape(1, 1))


def _block_diag(a):
    z = jnp.zeros((_DH, _DH), jnp.float32)
    return jnp.concatenate(
        [
            jnp.concatenate([a[0], z], axis=1),
            jnp.concatenate([z, a[1]], axis=1),
        ],
        axis=0,
    )


def _fold_layer(lp):
    folded = {}
    for t in _TYPES:
        rels_from = [r for r, (st, _) in _EDGE_DEFS.items() if st == t]
        wks, bks, wvs, bvs = [], [], [], []
        for r in rels_from:
            bd_a = _block_diag(lp["rel"][r]["a_rel"])
            bd_m = _block_diag(lp["rel"][r]["m_rel"])
            p = lp["rel"][r]["p_rel"] / np.float32(np.sqrt(_DH))
            ps = jnp.concatenate(
                [jnp.full((_DH,), 1.0) * p[0], jnp.full((_DH,), 1.0) * p[1]]
            )
            wks.append((lp["k"][t]["W"] @ bd_a) * ps[None, :])
            bks.append((lp["k"][t]["b"] @ bd_a) * ps)
            wv = lp["v"][t]["W"] @ bd_m
            bv = lp["v"][t]["b"] @ bd_m
            wvs.append(
                jnp.concatenate(
                    [wv, jnp.zeros((_D_H, _VW - _D_H), jnp.float32)], axis=1
                )
            )
            bvs.append(
                jnp.concatenate(
                    [
                        bv,
                        jnp.ones((2,), jnp.float32),
                        jnp.zeros((_VW - _D_H - 2,), jnp.float32),
                    ]
                )
            )
        wcat = jnp.concatenate(wks + wvs + [lp["q"][t]["W"]], axis=1)
        bcat = jnp.concatenate(bks + bvs + [lp["q"][t]["b"]])
        folded[t] = (rels_from, wcat, bcat)
    return folded


def _edge_phase_xla(kt, vt, q, edges, n_dst_map):
    agg = {}
    for dt_, rels in _DST_GROUPS.items():
        n_dst = n_dst_map[dt_]
        contribs = []
        dsts = []
        for r in rels:
            src = edges[r][0]
            dst = edges[r][1]
            km = kt[r][src]
            qe = q[dt_][dst]
            a2 = (km * qe).reshape(-1, _HEADS, _DH).sum(-1)
            w = jnp.exp(a2)
            vm = vt[r][src]
            scale = jnp.concatenate(
                [
                    jnp.repeat(w[:, 0:1], _DH, axis=1),
                    jnp.repeat(w[:, 1:2], _DH, axis=1),
                    w,
                    jnp.zeros((w.shape[0], _VW - _D_H - 2), jnp.float32),
                ],
                axis=1,
            )
            contribs.append(vm * scale)
            dsts.append(dst)
        c = jnp.concatenate(contribs, axis=0)
        d = jnp.concatenate(dsts, axis=0)
        agg[dt_] = jax.ops.segment_sum(c, d, num_segments=n_dst)
    return agg


def kernel(x_transaction, x_card, x_device, tx_time_decay, params,
           edge_tc, edge_td, edge_ct, edge_dt):
    edges = {"tc": edge_tc, "td": edge_td, "ct": edge_ct, "dt": edge_dt}
    n_dst_map = {
        "transaction": x_transaction.shape[0],
        "card": x_card.shape[0],
        "device": x_device.shape[0],
    }

    nl = params["node_lin"]
    h = {
        "transaction": _nl_tx(
            x_transaction, nl["transaction"]["W"], nl["transaction"]["b"],
            tx_time_decay,
        ),
        "card": _nl_scalar(x_card, nl["card"]["W"], nl["card"]["b"]),
        "device": _nl_scalar(x_device, nl["device"]["W"], nl["device"]["b"]),
    }

    for lp in params["convs"]:
        folded = _fold_layer(lp)
        kt, vt, q = {}, {}, {}
        for t in _TYPES:
            rels_from, wcat, bcat = folded[t]
            proj = _mm(h[t], wcat, bcat)
            nr = len(rels_from)
            for i, r in enumerate(rels_from):
                kt[r] = proj[:, i * _D_H : (i + 1) * _D_H]
                vt[r] = proj[:, nr * _D_H + i * _VW : nr * _D_H + (i + 1) * _VW]
            q[t] = proj[:, nr * (_D_H + _VW) :]

        agg = _edge_phase_xla(kt, vt, q, edges, n_dst_map)

        for t in _TYPES:
            h[t] = _attn_out(
                agg[t], h[t], lp["a"][t]["W"], lp["a"][t]["b"],
                jax.nn.sigmoid(lp["skip"][t]),
            )

    return _mm(h["transaction"], params["cls"]["W"], params["cls"]["b"])
